# baseline (speedup 1.0000x reference)
import functools

import jax
import jax.numpy as jnp
from jax import lax
from jax.experimental import pallas as pl
from jax.experimental.pallas import tpu as pltpu
from jax.experimental.pallas import tpu_sc as plsc

N = 10000
E = 320000
DP = 32
WD = 1
NC = 2
NS = 16
NW = NC * NS
K = 128
NCHUNK = 80
RING = 4
EPAD = NW * NCHUNK * K
NPAD = 12000
NG = 4
RPG = NPAD // NG
RPM = NPAD // NS
KM = 125


def _merge_planes(acc, ridx_v, mbuf, s):
    base = s * RPM
    for p in range(1, NG):
        pltpu.sync_copy(acc.at[pl.ds(p * NPAD + base, RPM)], mbuf)
        for cch in range(RPM // KM):
            pltpu.sync_copy(mbuf.at[pl.ds(cch * KM, KM)],
                            acc.at[ridx_v.at[cch]], add=True)


def _sc_deg_body(dst_hbm, zeros_hbm, ones_hbm, ridx_hbm, out_hbm,
                 dst_v, ones_v, ridx_v, mbuf, acc, dsem):
    c = lax.axis_index("c")
    s = lax.axis_index("s")
    wid = s * NC + c
    own = (s // NG) * NPAD + (s % NG) * RPG
    pltpu.sync_copy(dst_hbm.at[wid], dst_v)
    pltpu.sync_copy(ones_hbm, ones_v)
    pltpu.sync_copy(ridx_hbm.at[s], ridx_v)
    pltpu.sync_copy(zeros_hbm, acc.at[pl.ds(own, RPG)])
    plsc.subcore_barrier()

    def step(j, carry):
        pltpu.async_copy(ones_v, acc.at[dst_v.at[j]], dsem, add=True)

        @pl.when(j >= 8)
        def _():
            pltpu.make_async_copy(ones_v, acc.at[dst_v.at[j - 8]], dsem).wait()

        return carry

    lax.fori_loop(0, NCHUNK, step, 0)

    def drain(j, carry):
        pltpu.make_async_copy(ones_v, acc.at[dst_v.at[j]], dsem).wait()
        return carry

    lax.fori_loop(NCHUNK - 8, NCHUNK, drain, 0)
    plsc.subcore_barrier()
    _merge_planes(acc, ridx_v, mbuf, s)
    plsc.subcore_barrier()
    pltpu.sync_copy(acc.at[pl.ds(s * RPM, RPM)],
                    out_hbm.at[pl.ds(c * NPAD + s * RPM, RPM)])


def _sc_agg_body(g_hbm, src_hbm, dst_hbm, zeros_hbm, ridx_hbm, out_hbm,
                 src_v, dst_v, ridx_v, rows, mbuf, acc, gsems, ssems):
    c = lax.axis_index("c")
    s = lax.axis_index("s")
    wid = s * NC + c
    own = (s // NG) * NPAD + (s % NG) * RPG
    pltpu.sync_copy(src_hbm.at[wid], src_v)
    pltpu.sync_copy(dst_hbm.at[wid], dst_v)
    pltpu.sync_copy(ridx_hbm.at[s], ridx_v)
    pltpu.sync_copy(zeros_hbm, acc.at[pl.ds(own, RPG)])
    plsc.subcore_barrier()

    for b in range(RING):
        pltpu.async_copy(g_hbm.at[src_v.at[b]], rows.at[b], gsems.at[b])

    def step(i, carry):
        j0 = i * RING
        for b in range(RING):
            j = j0 + b
            pltpu.make_async_copy(g_hbm.at[src_v.at[j]], rows.at[b],
                                  gsems.at[b]).wait()
            pltpu.async_copy(rows.at[b], acc.at[dst_v.at[j]], ssems.at[b],
                             add=True)
        for b in range(RING):
            j = j0 + b

            @pl.when(j + RING < NCHUNK)
            def _():
                pltpu.make_async_copy(rows.at[b], acc.at[dst_v.at[j]],
                                      ssems.at[b]).wait()
                pltpu.async_copy(g_hbm.at[src_v.at[j + RING]], rows.at[b],
                                 gsems.at[b])

        return carry

    lax.fori_loop(0, NCHUNK // RING, step, 0)
    for b in range(RING):
        pltpu.make_async_copy(rows.at[b], acc.at[dst_v.at[NCHUNK - RING + b]],
                              ssems.at[b]).wait()
    plsc.subcore_barrier()
    _merge_planes(acc, ridx_v, mbuf, s)
    plsc.subcore_barrier()
    pltpu.sync_copy(acc.at[pl.ds(s * RPM, RPM)],
                    out_hbm.at[pl.ds(c * NPAD + s * RPM, RPM)])


@functools.cache
def _sc_kernels():
    mesh = plsc.VectorSubcoreMesh(core_axis_name="c", subcore_axis_name="s")
    params = pltpu.CompilerParams(use_tc_tiling_on_sc=False)
    sc_deg = pl.kernel(
        _sc_deg_body,
        out_type=jax.ShapeDtypeStruct((NC * NPAD, WD), jnp.float32),
        mesh=mesh,
        compiler_params=params,
        scratch_types=[
            pltpu.VMEM((NCHUNK, K), jnp.int32),
            pltpu.VMEM((K, WD), jnp.float32),
            pltpu.VMEM((RPM // KM, KM), jnp.int32),
            pltpu.VMEM((RPM, WD), jnp.float32),
            pltpu.VMEM_SHARED((NG * NPAD, WD), jnp.float32),
            pltpu.SemaphoreType.DMA,
        ],
    )
    sc_agg = pl.kernel(
        _sc_agg_body,
        out_type=jax.ShapeDtypeStruct((NC * NPAD, DP), jnp.bfloat16),
        mesh=mesh,
        compiler_params=params,
        scratch_types=[
            pltpu.VMEM((NCHUNK, K), jnp.int32),
            pltpu.VMEM((NCHUNK, K), jnp.int32),
            pltpu.VMEM((RPM // KM, KM), jnp.int32),
            pltpu.VMEM((RING, K, DP), jnp.bfloat16),
            pltpu.VMEM((RPM, DP), jnp.bfloat16),
            pltpu.VMEM_SHARED((NG * NPAD, DP), jnp.bfloat16),
            pltpu.SemaphoreType.DMA((RING,)),
            pltpu.SemaphoreType.DMA((RING,)),
        ],
    )
    return sc_deg, sc_agg


_NB = 5
_BR = N // _NB
_PB = NPAD // _BR

_deg0_spec = pl.BlockSpec((_BR, WD), lambda i: (i, 0))
_deg1_spec = pl.BlockSpec((_BR, WD), lambda i: (_PB + i, 0))
_agg0_spec = pl.BlockSpec((_BR, DP), lambda i: (i, 0))
_agg1_spec = pl.BlockSpec((_BR, DP), lambda i: (_PB + i, 0))
_row_spec = pl.BlockSpec((_BR, DP), lambda i: (i, 0))


def _dinv_of(d0_ref, d1_ref):
    return lax.rsqrt(d0_ref[:, 0:1] + d1_ref[:, 0:1] + 1.0)


def _agg_of(a0_ref, a1_ref, g_ref):
    return (a0_ref[...].astype(jnp.float32) + a1_ref[...].astype(jnp.float32)
            + g_ref[...].astype(jnp.float32))


def _tcmm_body(x_ref, w_ref, o_ref):
    o_ref[...] = jnp.dot(x_ref[...], w_ref[...],
                         preferred_element_type=jnp.float32)


def _tc1_body(d0_ref, d1_ref, h_ref, o_ref):
    dinv = _dinv_of(d0_ref, d1_ref)
    o_ref[...] = (h_ref[...] * dinv).astype(jnp.bfloat16)


def _tc2_body(d0_ref, d1_ref, a0_ref, a1_ref, g_ref, w_ref, b_ref, o_ref):
    dinv = _dinv_of(d0_ref, d1_ref)
    z = jnp.maximum(_agg_of(a0_ref, a1_ref, g_ref) * dinv + b_ref[...], 0.0)
    o_ref[...] = (jnp.dot(z, w_ref[...], preferred_element_type=jnp.float32)
                  * dinv).astype(jnp.bfloat16)


def _tc3_body(d0_ref, d1_ref, a0_ref, a1_ref, g_ref, b_ref, o_ref):
    dinv = _dinv_of(d0_ref, d1_ref)
    o_ref[...] = _agg_of(a0_ref, a1_ref, g_ref) * dinv + b_ref[...]


def _tcmm(x, w1p):
    return pl.pallas_call(
        _tcmm_body,
        grid=(_NB,),
        in_specs=[pl.BlockSpec((_BR, 128), lambda i: (i, 0)),
                  pl.BlockSpec((128, DP), lambda i: (0, 0))],
        out_specs=_row_spec,
        out_shape=jax.ShapeDtypeStruct((N, DP), jnp.float32),
    )(x, w1p)


def _tc1(degp, h):
    return pl.pallas_call(
        _tc1_body,
        grid=(_NB,),
        in_specs=[_deg0_spec, _deg1_spec, _row_spec],
        out_specs=_row_spec,
        out_shape=jax.ShapeDtypeStruct((N, DP), jnp.bfloat16),
    )(degp, degp, h)


def _tc2(degp, aggp, g1, w2p, b1p):
    return pl.pallas_call(
        _tc2_body,
        grid=(_NB,),
        in_specs=[_deg0_spec, _deg1_spec, _agg0_spec, _agg1_spec, _row_spec,
                  pl.BlockSpec((DP, DP), lambda i: (0, 0)),
                  pl.BlockSpec((1, DP), lambda i: (0, 0))],
        out_specs=_row_spec,
        out_shape=jax.ShapeDtypeStruct((N, DP), jnp.bfloat16),
    )(degp, degp, aggp, aggp, g1, w2p, b1p)


def _tc3(degp, aggp, g2, b2p):
    return pl.pallas_call(
        _tc3_body,
        grid=(_NB,),
        in_specs=[_deg0_spec, _deg1_spec, _agg0_spec, _agg1_spec, _row_spec,
                  pl.BlockSpec((1, DP), lambda i: (0, 0))],
        out_specs=_row_spec,
        out_shape=jax.ShapeDtypeStruct((N, DP), jnp.float32),
    )(degp, degp, aggp, aggp, g2, b2p)


def kernel(x, edge_index, W1, b1, W2, b2):
    ei = edge_index.astype(jnp.int32)
    src = jnp.concatenate([ei[0], jnp.zeros((EPAD - E,), jnp.int32)])
    dst = jnp.concatenate([ei[1], jnp.full((EPAD - E,), N, jnp.int32)])
    srcI = src.reshape(NW, NCHUNK, K)
    grp_off = (jnp.arange(NW, dtype=jnp.int32)[:, None, None] // 8) * NPAD
    dstI = dst.reshape(NW, NCHUNK, K) + grp_off

    w1p = jnp.zeros((128, DP), jnp.float32).at[:, :30].set(W1)
    b1p = jnp.zeros((1, DP), jnp.float32).at[0, :30].set(b1)
    w2p = jnp.zeros((DP, DP), jnp.float32).at[:30, :25].set(W2)
    b2p = jnp.zeros((1, DP), jnp.float32).at[0, :25].set(b2)

    zdeg = jnp.zeros((RPG, WD), jnp.float32)
    zagg = jnp.zeros((RPG, DP), jnp.bfloat16)
    ones16 = jnp.ones((K, WD), jnp.float32)
    ridx = (jnp.arange(NS, dtype=jnp.int32)[:, None, None] * RPM
            + jnp.arange(RPM // KM, dtype=jnp.int32)[None, :, None] * KM
            + jnp.arange(KM, dtype=jnp.int32)[None, None, :])

    sc_deg, sc_agg = _sc_kernels()
    h1 = _tcmm(x, w1p)
    degp = sc_deg(dstI, zdeg, ones16, ridx)
    g1 = _tc1(degp, h1)
    a1 = sc_agg(g1, srcI, dstI, zagg, ridx)
    g2 = _tc2(degp, a1, g1, w2p, b1p)
    a2 = sc_agg(g2, srcI, dstI, zagg, ridx)
    out = _tc3(degp, a2, g2, b2p)
    return out[:, :25]


# speedup vs baseline: 1.0197x; 1.0197x over previous
import functools

import jax
import jax.numpy as jnp
from jax import lax
from jax.experimental import pallas as pl
from jax.experimental.pallas import tpu as pltpu
from jax.experimental.pallas import tpu_sc as plsc

N = 10000
E = 320000
DP = 32
WD = 16
NC = 2
NS = 16
NW = NC * NS
K = 128
NCHUNK = 80
RING = 4
EPAD = NW * NCHUNK * K
NPAD = 12000
NG = 4
RPG = NPAD // NG
RPM = NPAD // NS
KM = 125


def _merge_planes(acc, ridx_v, mbuf, s):
    base = s * RPM
    for p in range(1, NG):
        pltpu.sync_copy(acc.at[pl.ds(p * NPAD + base, RPM)], mbuf)
        for cch in range(RPM // KM):
            pltpu.sync_copy(mbuf.at[pl.ds(cch * KM, KM)],
                            acc.at[ridx_v.at[cch]], add=True)


def _sc_deg_body(dst_hbm, zeros_hbm, ones_hbm, ridx_hbm, out_hbm,
                 dst_v, ones_v, ridx_v, mbuf, acc, dsem):
    c = lax.axis_index("c")
    s = lax.axis_index("s")
    wid = s * NC + c
    own = (s // NG) * NPAD + (s % NG) * RPG
    pltpu.sync_copy(dst_hbm.at[wid], dst_v)
    pltpu.sync_copy(ones_hbm, ones_v)
    pltpu.sync_copy(ridx_hbm.at[s], ridx_v)
    pltpu.sync_copy(zeros_hbm, acc.at[pl.ds(own, RPG)])
    plsc.subcore_barrier()

    def step(j, carry):
        pltpu.async_copy(ones_v, acc.at[dst_v.at[j]], dsem, add=True)

        @pl.when(j >= 8)
        def _():
            pltpu.make_async_copy(ones_v, acc.at[dst_v.at[j - 8]], dsem).wait()

        return carry

    lax.fori_loop(0, NCHUNK, step, 0)

    def drain(j, carry):
        pltpu.make_async_copy(ones_v, acc.at[dst_v.at[j]], dsem).wait()
        return carry

    lax.fori_loop(NCHUNK - 8, NCHUNK, drain, 0)
    plsc.subcore_barrier()
    _merge_planes(acc, ridx_v, mbuf, s)
    plsc.subcore_barrier()
    pltpu.sync_copy(acc.at[pl.ds(s * RPM, RPM)],
                    out_hbm.at[pl.ds(c * NPAD + s * RPM, RPM)])


def _sc_agg_body(g_hbm, src_hbm, dst_hbm, zeros_hbm, ridx_hbm, out_hbm,
                 src_v, dst_v, ridx_v, rows, mbuf, acc, gsems, ssems):
    c = lax.axis_index("c")
    s = lax.axis_index("s")
    wid = s * NC + c
    own = (s // NG) * NPAD + (s % NG) * RPG
    pltpu.sync_copy(src_hbm.at[wid], src_v)
    pltpu.sync_copy(dst_hbm.at[wid], dst_v)
    pltpu.sync_copy(ridx_hbm.at[s], ridx_v)
    pltpu.sync_copy(zeros_hbm, acc.at[pl.ds(own, RPG)])
    plsc.subcore_barrier()

    for b in range(RING):
        pltpu.async_copy(g_hbm.at[src_v.at[b]], rows.at[b], gsems.at[b])

    def step(i, carry):
        j0 = i * RING
        for b in range(RING):
            j = j0 + b
            pltpu.make_async_copy(g_hbm.at[src_v.at[j]], rows.at[b],
                                  gsems.at[b]).wait()
            pltpu.async_copy(rows.at[b], acc.at[dst_v.at[j]], ssems.at[b],
                             add=True)
        for b in range(RING):
            j = j0 + b

            @pl.when(j + RING < NCHUNK)
            def _():
                pltpu.make_async_copy(rows.at[b], acc.at[dst_v.at[j]],
                                      ssems.at[b]).wait()
                pltpu.async_copy(g_hbm.at[src_v.at[j + RING]], rows.at[b],
                                 gsems.at[b])

        return carry

    lax.fori_loop(0, NCHUNK // RING, step, 0)
    for b in range(RING):
        pltpu.make_async_copy(rows.at[b], acc.at[dst_v.at[NCHUNK - RING + b]],
                              ssems.at[b]).wait()
    plsc.subcore_barrier()
    _merge_planes(acc, ridx_v, mbuf, s)
    plsc.subcore_barrier()
    pltpu.sync_copy(acc.at[pl.ds(s * RPM, RPM)],
                    out_hbm.at[pl.ds(c * NPAD + s * RPM, RPM)])


@functools.cache
def _sc_kernels():
    mesh = plsc.VectorSubcoreMesh(core_axis_name="c", subcore_axis_name="s")
    params = pltpu.CompilerParams(use_tc_tiling_on_sc=False)
    sc_deg = pl.kernel(
        _sc_deg_body,
        out_type=jax.ShapeDtypeStruct((NC * NPAD, WD), jnp.float32),
        mesh=mesh,
        compiler_params=params,
        scratch_types=[
            pltpu.VMEM((NCHUNK, K), jnp.int32),
            pltpu.VMEM((K, WD), jnp.float32),
            pltpu.VMEM((RPM // KM, KM), jnp.int32),
            pltpu.VMEM((RPM, WD), jnp.float32),
            pltpu.VMEM_SHARED((NG * NPAD, WD), jnp.float32),
            pltpu.SemaphoreType.DMA,
        ],
    )
    sc_agg = pl.kernel(
        _sc_agg_body,
        out_type=jax.ShapeDtypeStruct((NC * NPAD, DP), jnp.bfloat16),
        mesh=mesh,
        compiler_params=params,
        scratch_types=[
            pltpu.VMEM((NCHUNK, K), jnp.int32),
            pltpu.VMEM((NCHUNK, K), jnp.int32),
            pltpu.VMEM((RPM // KM, KM), jnp.int32),
            pltpu.VMEM((RING, K, DP), jnp.bfloat16),
            pltpu.VMEM((RPM, DP), jnp.bfloat16),
            pltpu.VMEM_SHARED((NG * NPAD, DP), jnp.bfloat16),
            pltpu.SemaphoreType.DMA((RING,)),
            pltpu.SemaphoreType.DMA((RING,)),
        ],
    )
    return sc_deg, sc_agg


_NB = 5
_BR = N // _NB
_PB = NPAD // _BR

_deg0_spec = pl.BlockSpec((_BR, WD), lambda i: (i, 0))
_deg1_spec = pl.BlockSpec((_BR, WD), lambda i: (_PB + i, 0))
_agg0_spec = pl.BlockSpec((_BR, DP), lambda i: (i, 0))
_agg1_spec = pl.BlockSpec((_BR, DP), lambda i: (_PB + i, 0))
_row_spec = pl.BlockSpec((_BR, DP), lambda i: (i, 0))


def _dinv_of(d0_ref, d1_ref):
    return lax.rsqrt(d0_ref[:, 0:1] + d1_ref[:, 0:1] + 1.0)


def _agg_of(a0_ref, a1_ref, g_ref):
    return (a0_ref[...].astype(jnp.float32) + a1_ref[...].astype(jnp.float32)
            + g_ref[...].astype(jnp.float32))


def _tcmm_body(x_ref, w_ref, o_ref):
    o_ref[...] = jnp.dot(x_ref[...], w_ref[...],
                         preferred_element_type=jnp.float32)


def _tc1_body(d0_ref, d1_ref, h_ref, o_ref):
    dinv = _dinv_of(d0_ref, d1_ref)
    o_ref[...] = (h_ref[...] * dinv).astype(jnp.bfloat16)


def _tc2_body(d0_ref, d1_ref, a0_ref, a1_ref, g_ref, w_ref, b_ref, o_ref):
    dinv = _dinv_of(d0_ref, d1_ref)
    z = jnp.maximum(_agg_of(a0_ref, a1_ref, g_ref) * dinv + b_ref[...], 0.0)
    o_ref[...] = (jnp.dot(z, w_ref[...], preferred_element_type=jnp.float32)
                  * dinv).astype(jnp.bfloat16)


def _tc3_body(d0_ref, d1_ref, a0_ref, a1_ref, g_ref, b_ref, o_ref):
    dinv = _dinv_of(d0_ref, d1_ref)
    o_ref[...] = _agg_of(a0_ref, a1_ref, g_ref) * dinv + b_ref[...]


def _tcmm(x, w1p):
    return pl.pallas_call(
        _tcmm_body,
        grid=(_NB,),
        in_specs=[pl.BlockSpec((_BR, 128), lambda i: (i, 0)),
                  pl.BlockSpec((128, DP), lambda i: (0, 0))],
        out_specs=_row_spec,
        out_shape=jax.ShapeDtypeStruct((N, DP), jnp.float32),
    )(x, w1p)


def _tc1(degp, h):
    return pl.pallas_call(
        _tc1_body,
        grid=(_NB,),
        in_specs=[_deg0_spec, _deg1_spec, _row_spec],
        out_specs=_row_spec,
        out_shape=jax.ShapeDtypeStruct((N, DP), jnp.bfloat16),
    )(degp, degp, h)


def _tc2(degp, aggp, g1, w2p, b1p):
    return pl.pallas_call(
        _tc2_body,
        grid=(_NB,),
        in_specs=[_deg0_spec, _deg1_spec, _agg0_spec, _agg1_spec, _row_spec,
                  pl.BlockSpec((DP, DP), lambda i: (0, 0)),
                  pl.BlockSpec((1, DP), lambda i: (0, 0))],
        out_specs=_row_spec,
        out_shape=jax.ShapeDtypeStruct((N, DP), jnp.bfloat16),
    )(degp, degp, aggp, aggp, g1, w2p, b1p)


def _tc3(degp, aggp, g2, b2p):
    return pl.pallas_call(
        _tc3_body,
        grid=(_NB,),
        in_specs=[_deg0_spec, _deg1_spec, _agg0_spec, _agg1_spec, _row_spec,
                  pl.BlockSpec((1, DP), lambda i: (0, 0))],
        out_specs=_row_spec,
        out_shape=jax.ShapeDtypeStruct((N, DP), jnp.float32),
    )(degp, degp, aggp, aggp, g2, b2p)


def kernel(x, edge_index, W1, b1, W2, b2):
    ei = edge_index.astype(jnp.int32)
    src = jnp.concatenate([ei[0], jnp.zeros((EPAD - E,), jnp.int32)])
    dst = jnp.concatenate([ei[1], jnp.full((EPAD - E,), N, jnp.int32)])
    srcI = src.reshape(NW, NCHUNK, K)
    grp_off = (jnp.arange(NW, dtype=jnp.int32)[:, None, None] // 8) * NPAD
    dstI = dst.reshape(NW, NCHUNK, K) + grp_off

    w1p = jnp.zeros((128, DP), jnp.float32).at[:, :30].set(W1)
    b1p = jnp.zeros((1, DP), jnp.float32).at[0, :30].set(b1)
    w2p = jnp.zeros((DP, DP), jnp.float32).at[:30, :25].set(W2)
    b2p = jnp.zeros((1, DP), jnp.float32).at[0, :25].set(b2)

    zdeg = jnp.zeros((RPG, WD), jnp.float32)
    zagg = jnp.zeros((RPG, DP), jnp.bfloat16)
    ones16 = jnp.ones((K, WD), jnp.float32)
    ridx = (jnp.arange(NS, dtype=jnp.int32)[:, None, None] * RPM
            + jnp.arange(RPM // KM, dtype=jnp.int32)[None, :, None] * KM
            + jnp.arange(KM, dtype=jnp.int32)[None, None, :])

    sc_deg, sc_agg = _sc_kernels()
    h1 = _tcmm(x, w1p)
    degp = sc_deg(dstI, zdeg, ones16, ridx)
    g1 = _tc1(degp, h1)
    a1 = sc_agg(g1, srcI, dstI, zagg, ridx)
    g2 = _tc2(degp, a1, g1, w2p, b1p)
    a2 = sc_agg(g2, srcI, dstI, zagg, ridx)
    out = _tc3(degp, a2, g2, b2p)
    return out[:, :25]


# speedup vs baseline: 1.4048x; 1.3776x over previous
import functools

import jax
import jax.numpy as jnp
from jax import lax
from jax.experimental import pallas as pl
from jax.experimental.pallas import tpu as pltpu
from jax.experimental.pallas import tpu_sc as plsc

N = 10000
E = 320000
DP = 32
WD = 16
NC = 2
NS = 16
NW = NC * NS
K = 125
NCHUNK = 80
RING = 4
NPAD = 12000
NG = 4
RPG = NPAD // NG
RPM = NPAD // NS
KM = 125


def _merge_planes(acc, ridx_v, mbuf, s):
    base = s * RPM
    for p in range(1, NG):
        pltpu.sync_copy(acc.at[pl.ds(p * NPAD + base, RPM)], mbuf)
        for cch in range(RPM // KM):
            pltpu.sync_copy(mbuf.at[pl.ds(cch * KM, KM)],
                            acc.at[ridx_v.at[cch]], add=True)


def _sc_deg_body(dst_hbm, zeros_hbm, ones_hbm, ridx_hbm, out_hbm,
                 dst_v, ones_v, ridx_v, mbuf, acc, dsem):
    c = lax.axis_index("c")
    s = lax.axis_index("s")
    wid = s * NC + c
    own = (s // NG) * NPAD + (s % NG) * RPG
    pltpu.sync_copy(dst_hbm.at[wid], dst_v)
    pltpu.sync_copy(ones_hbm, ones_v)
    pltpu.sync_copy(ridx_hbm.at[s], ridx_v)
    pltpu.sync_copy(zeros_hbm, acc.at[pl.ds(own, RPG)])
    plsc.subcore_barrier()

    def step(j, carry):
        pltpu.async_copy(ones_v, acc.at[dst_v.at[j]], dsem, add=True)

        @pl.when(j >= 8)
        def _():
            pltpu.make_async_copy(ones_v, acc.at[dst_v.at[j - 8]], dsem).wait()

        return carry

    lax.fori_loop(0, NCHUNK, step, 0)

    def drain(j, carry):
        pltpu.make_async_copy(ones_v, acc.at[dst_v.at[j]], dsem).wait()
        return carry

    lax.fori_loop(NCHUNK - 8, NCHUNK, drain, 0)
    plsc.subcore_barrier()
    _merge_planes(acc, ridx_v, mbuf, s)
    plsc.subcore_barrier()
    pltpu.sync_copy(acc.at[pl.ds(s * RPM, RPM)],
                    out_hbm.at[pl.ds(c * NPAD + s * RPM, RPM)])


def _sc_agg_body(g_hbm, src_hbm, dst_hbm, zeros_hbm, ridx_hbm, out_hbm,
                 src_v, dst_v, ridx_v, rows, mbuf, acc, gsems, ssems):
    c = lax.axis_index("c")
    s = lax.axis_index("s")
    wid = s * NC + c
    own = (s // NG) * NPAD + (s % NG) * RPG
    pltpu.sync_copy(src_hbm.at[wid], src_v)
    pltpu.sync_copy(dst_hbm.at[wid], dst_v)
    pltpu.sync_copy(ridx_hbm.at[s], ridx_v)
    pltpu.sync_copy(zeros_hbm, acc.at[pl.ds(own, RPG)])
    plsc.subcore_barrier()

    for b in range(RING):
        pltpu.async_copy(g_hbm.at[src_v.at[b]], rows.at[b], gsems.at[b])

    def step(i, carry):
        j0 = i * RING
        for b in range(RING):
            j = j0 + b
            pltpu.make_async_copy(g_hbm.at[src_v.at[j]], rows.at[b],
                                  gsems.at[b]).wait()
            pltpu.async_copy(rows.at[b], acc.at[dst_v.at[j]], ssems.at[b],
                             add=True)
        for b in range(RING):
            j = j0 + b

            @pl.when(j + RING < NCHUNK)
            def _():
                pltpu.make_async_copy(rows.at[b], acc.at[dst_v.at[j]],
                                      ssems.at[b]).wait()
                pltpu.async_copy(g_hbm.at[src_v.at[j + RING]], rows.at[b],
                                 gsems.at[b])

        return carry

    lax.fori_loop(0, NCHUNK // RING, step, 0)
    for b in range(RING):
        pltpu.make_async_copy(rows.at[b], acc.at[dst_v.at[NCHUNK - RING + b]],
                              ssems.at[b]).wait()
    plsc.subcore_barrier()
    _merge_planes(acc, ridx_v, mbuf, s)
    plsc.subcore_barrier()
    pltpu.sync_copy(acc.at[pl.ds(s * RPM, RPM)],
                    out_hbm.at[pl.ds(c * NPAD + s * RPM, RPM)])


@functools.cache
def _sc_kernels():
    mesh = plsc.VectorSubcoreMesh(core_axis_name="c", subcore_axis_name="s")
    params = pltpu.CompilerParams(use_tc_tiling_on_sc=False)
    sc_deg = pl.kernel(
        _sc_deg_body,
        out_type=jax.ShapeDtypeStruct((NC * NPAD, WD), jnp.float32),
        mesh=mesh,
        compiler_params=params,
        scratch_types=[
            pltpu.VMEM((NCHUNK, K), jnp.int32),
            pltpu.VMEM((K, WD), jnp.float32),
            pltpu.VMEM((RPM // KM, KM), jnp.int32),
            pltpu.VMEM((RPM, WD), jnp.float32),
            pltpu.VMEM_SHARED((NG * NPAD, WD), jnp.float32),
            pltpu.SemaphoreType.DMA,
        ],
    )
    sc_agg = pl.kernel(
        _sc_agg_body,
        out_type=jax.ShapeDtypeStruct((NC * NPAD, DP), jnp.bfloat16),
        mesh=mesh,
        compiler_params=params,
        scratch_types=[
            pltpu.VMEM((NCHUNK, K), jnp.int32),
            pltpu.VMEM((NCHUNK, K), jnp.int32),
            pltpu.VMEM((RPM // KM, KM), jnp.int32),
            pltpu.VMEM((RING, K, DP), jnp.bfloat16),
            pltpu.VMEM((RPM, DP), jnp.bfloat16),
            pltpu.VMEM_SHARED((NG * NPAD, DP), jnp.bfloat16),
            pltpu.SemaphoreType.DMA((RING,)),
            pltpu.SemaphoreType.DMA((RING,)),
        ],
    )
    return sc_deg, sc_agg


_NB = 5
_BR = N // _NB
_PB = NPAD // _BR

_deg0_spec = pl.BlockSpec((_BR, WD), lambda i: (i, 0))
_deg1_spec = pl.BlockSpec((_BR, WD), lambda i: (_PB + i, 0))
_agg0_spec = pl.BlockSpec((_BR, DP), lambda i: (i, 0))
_agg1_spec = pl.BlockSpec((_BR, DP), lambda i: (_PB + i, 0))
_row_spec = pl.BlockSpec((_BR, DP), lambda i: (i, 0))


def _dinv_of(d0_ref, d1_ref):
    return lax.rsqrt(d0_ref[:, 0:1] + d1_ref[:, 0:1] + 1.0)


def _agg_of(a0_ref, a1_ref, g_ref):
    return (a0_ref[...].astype(jnp.float32) + a1_ref[...].astype(jnp.float32)
            + g_ref[...].astype(jnp.float32))


def _tcmm_body(x_ref, w_ref, o_ref):
    o_ref[...] = jnp.dot(x_ref[...], w_ref[...],
                         preferred_element_type=jnp.float32)


def _tc1_body(d0_ref, d1_ref, h_ref, o_ref):
    dinv = _dinv_of(d0_ref, d1_ref)
    o_ref[...] = (h_ref[...] * dinv).astype(jnp.bfloat16)


def _tc2_body(d0_ref, d1_ref, a0_ref, a1_ref, g_ref, w_ref, b_ref, o_ref):
    dinv = _dinv_of(d0_ref, d1_ref)
    z = jnp.maximum(_agg_of(a0_ref, a1_ref, g_ref) * dinv + b_ref[...], 0.0)
    o_ref[...] = (jnp.dot(z, w_ref[...], preferred_element_type=jnp.float32)
                  * dinv).astype(jnp.bfloat16)


def _tc3_body(d0_ref, d1_ref, a0_ref, a1_ref, g_ref, b_ref, o_ref):
    dinv = _dinv_of(d0_ref, d1_ref)
    o_ref[...] = (_agg_of(a0_ref, a1_ref, g_ref) * dinv + b_ref[...])[:, :25]


def _tcmm(x, w1p):
    return pl.pallas_call(
        _tcmm_body,
        grid=(_NB,),
        in_specs=[pl.BlockSpec((_BR, 128), lambda i: (i, 0)),
                  pl.BlockSpec((128, DP), lambda i: (0, 0))],
        out_specs=_row_spec,
        out_shape=jax.ShapeDtypeStruct((N, DP), jnp.float32),
    )(x, w1p)


def _tc1(degp, h):
    return pl.pallas_call(
        _tc1_body,
        grid=(_NB,),
        in_specs=[_deg0_spec, _deg1_spec, _row_spec],
        out_specs=_row_spec,
        out_shape=jax.ShapeDtypeStruct((N, DP), jnp.bfloat16),
    )(degp, degp, h)


def _tc2(degp, aggp, g1, w2p, b1p):
    return pl.pallas_call(
        _tc2_body,
        grid=(_NB,),
        in_specs=[_deg0_spec, _deg1_spec, _agg0_spec, _agg1_spec, _row_spec,
                  pl.BlockSpec((DP, DP), lambda i: (0, 0)),
                  pl.BlockSpec((1, DP), lambda i: (0, 0))],
        out_specs=_row_spec,
        out_shape=jax.ShapeDtypeStruct((N, DP), jnp.bfloat16),
    )(degp, degp, aggp, aggp, g1, w2p, b1p)


def _tc3(degp, aggp, g2, b2p):
    return pl.pallas_call(
        _tc3_body,
        grid=(_NB,),
        in_specs=[_deg0_spec, _deg1_spec, _agg0_spec, _agg1_spec, _row_spec,
                  pl.BlockSpec((1, DP), lambda i: (0, 0))],
        out_specs=pl.BlockSpec((_BR, 25), lambda i: (i, 0)),
        out_shape=jax.ShapeDtypeStruct((N, 25), jnp.float32),
    )(degp, degp, aggp, aggp, g2, b2p)


def kernel(x, edge_index, W1, b1, W2, b2):
    ei = edge_index.astype(jnp.int32)
    srcI = ei[0].reshape(NW, NCHUNK, K)
    grp_off = (jnp.arange(NW, dtype=jnp.int32)[:, None, None] // 8) * NPAD
    dstI = ei[1].reshape(NW, NCHUNK, K) + grp_off

    w1p = jnp.zeros((128, DP), jnp.float32).at[:, :30].set(W1)
    b1p = jnp.zeros((1, DP), jnp.float32).at[0, :30].set(b1)
    w2p = jnp.zeros((DP, DP), jnp.float32).at[:30, :25].set(W2)
    b2p = jnp.zeros((1, DP), jnp.float32).at[0, :25].set(b2)

    zdeg = jnp.zeros((RPG, WD), jnp.float32)
    zagg = jnp.zeros((RPG, DP), jnp.bfloat16)
    ones16 = jnp.ones((K, WD), jnp.float32)
    ridx = (jnp.arange(NS, dtype=jnp.int32)[:, None, None] * RPM
            + jnp.arange(RPM // KM, dtype=jnp.int32)[None, :, None] * KM
            + jnp.arange(KM, dtype=jnp.int32)[None, None, :])

    sc_deg, sc_agg = _sc_kernels()
    h1 = _tcmm(x, w1p)
    degp = sc_deg(dstI, zdeg, ones16, ridx)
    g1 = _tc1(degp, h1)
    a1 = sc_agg(g1, srcI, dstI, zagg, ridx)
    g2 = _tc2(degp, a1, g1, w2p, b1p)
    a2 = sc_agg(g2, srcI, dstI, zagg, ridx)
    return _tc3(degp, a2, g2, b2p)


# speedup vs baseline: 1.4059x; 1.0008x over previous
import functools

import jax
import jax.numpy as jnp
from jax import lax
from jax.experimental import pallas as pl
from jax.experimental.pallas import tpu as pltpu
from jax.experimental.pallas import tpu_sc as plsc

N = 10000
E = 320000
DP = 32
WD = 16
NC = 2
NS = 16
NW = NC * NS
K = 125
NCHUNK = 80
RING = 4
NPAD = 12000
NG = 4
RPG = NPAD // NG
RPM = NPAD // NS
KM = 125


def _merge_planes(acc, ridx_v, mbuf, s):
    base = s * RPM
    for p in range(1, NG):
        pltpu.sync_copy(acc.at[pl.ds(p * NPAD + base, RPM)], mbuf)
        for cch in range(RPM // KM):
            pltpu.sync_copy(mbuf.at[pl.ds(cch * KM, KM)],
                            acc.at[ridx_v.at[cch]], add=True)


def _sc_deg_body(dst_hbm, zeros_hbm, ones_hbm, ridx_hbm, out_hbm,
                 dst_v, ones_v, ridx_v, mbuf, acc, dsem):
    c = lax.axis_index("c")
    s = lax.axis_index("s")
    wid = s * NC + c
    own = (s // NG) * NPAD + (s % NG) * RPG
    pltpu.sync_copy(dst_hbm.at[wid], dst_v)
    pltpu.sync_copy(ones_hbm, ones_v)
    pltpu.sync_copy(ridx_hbm.at[s], ridx_v)
    pltpu.sync_copy(zeros_hbm, acc.at[pl.ds(own, RPG)])
    plsc.subcore_barrier()

    plane = acc.at[pl.ds((s // NG) * NPAD, NPAD)]

    def step(j, carry):
        pltpu.async_copy(ones_v, plane.at[dst_v.at[j]], dsem, add=True)

        @pl.when(j >= 8)
        def _():
            pltpu.make_async_copy(ones_v, plane.at[dst_v.at[j - 8]], dsem).wait()

        return carry

    lax.fori_loop(0, NCHUNK, step, 0)

    def drain(j, carry):
        pltpu.make_async_copy(ones_v, plane.at[dst_v.at[j]], dsem).wait()
        return carry

    lax.fori_loop(NCHUNK - 8, NCHUNK, drain, 0)
    plsc.subcore_barrier()
    _merge_planes(acc, ridx_v, mbuf, s)
    plsc.subcore_barrier()
    pltpu.sync_copy(acc.at[pl.ds(s * RPM, RPM)],
                    out_hbm.at[pl.ds(c * NPAD + s * RPM, RPM)])


def _sc_agg_body(g_hbm, src_hbm, dst_hbm, zeros_hbm, ridx_hbm, out_hbm,
                 src_v, dst_v, ridx_v, rows, mbuf, acc, gsems, ssems):
    c = lax.axis_index("c")
    s = lax.axis_index("s")
    wid = s * NC + c
    own = (s // NG) * NPAD + (s % NG) * RPG
    pltpu.sync_copy(src_hbm.at[wid], src_v)
    pltpu.sync_copy(dst_hbm.at[wid], dst_v)
    pltpu.sync_copy(ridx_hbm.at[s], ridx_v)
    pltpu.sync_copy(zeros_hbm, acc.at[pl.ds(own, RPG)])
    plsc.subcore_barrier()

    plane = acc.at[pl.ds((s // NG) * NPAD, NPAD)]
    for b in range(RING):
        pltpu.async_copy(g_hbm.at[src_v.at[b]], rows.at[b], gsems.at[b])

    def step(i, carry):
        j0 = i * RING
        for b in range(RING):
            j = j0 + b
            pltpu.make_async_copy(g_hbm.at[src_v.at[j]], rows.at[b],
                                  gsems.at[b]).wait()
            pltpu.async_copy(rows.at[b], plane.at[dst_v.at[j]], ssems.at[b],
                             add=True)
        for b in range(RING):
            j = j0 + b

            @pl.when(j + RING < NCHUNK)
            def _():
                pltpu.make_async_copy(rows.at[b], plane.at[dst_v.at[j]],
                                      ssems.at[b]).wait()
                pltpu.async_copy(g_hbm.at[src_v.at[j + RING]], rows.at[b],
                                 gsems.at[b])

        return carry

    lax.fori_loop(0, NCHUNK // RING, step, 0)
    for b in range(RING):
        pltpu.make_async_copy(rows.at[b], plane.at[dst_v.at[NCHUNK - RING + b]],
                              ssems.at[b]).wait()
    plsc.subcore_barrier()
    _merge_planes(acc, ridx_v, mbuf, s)
    plsc.subcore_barrier()
    pltpu.sync_copy(acc.at[pl.ds(s * RPM, RPM)],
                    out_hbm.at[pl.ds(c * NPAD + s * RPM, RPM)])


@functools.cache
def _sc_kernels():
    mesh = plsc.VectorSubcoreMesh(core_axis_name="c", subcore_axis_name="s")
    params = pltpu.CompilerParams(use_tc_tiling_on_sc=False)
    sc_deg = pl.kernel(
        _sc_deg_body,
        out_type=jax.ShapeDtypeStruct((NC * NPAD, WD), jnp.float32),
        mesh=mesh,
        compiler_params=params,
        scratch_types=[
            pltpu.VMEM((NCHUNK, K), jnp.int32),
            pltpu.VMEM((K, WD), jnp.float32),
            pltpu.VMEM((RPM // KM, KM), jnp.int32),
            pltpu.VMEM((RPM, WD), jnp.float32),
            pltpu.VMEM_SHARED((NG * NPAD, WD), jnp.float32),
            pltpu.SemaphoreType.DMA,
        ],
    )
    sc_agg = pl.kernel(
        _sc_agg_body,
        out_type=jax.ShapeDtypeStruct((NC * NPAD, DP), jnp.bfloat16),
        mesh=mesh,
        compiler_params=params,
        scratch_types=[
            pltpu.VMEM((NCHUNK, K), jnp.int32),
            pltpu.VMEM((NCHUNK, K), jnp.int32),
            pltpu.VMEM((RPM // KM, KM), jnp.int32),
            pltpu.VMEM((RING, K, DP), jnp.bfloat16),
            pltpu.VMEM((RPM, DP), jnp.bfloat16),
            pltpu.VMEM_SHARED((NG * NPAD, DP), jnp.bfloat16),
            pltpu.SemaphoreType.DMA((RING,)),
            pltpu.SemaphoreType.DMA((RING,)),
        ],
    )
    return sc_deg, sc_agg


_NB = 5
_BR = N // _NB
_PB = NPAD // _BR

_deg0_spec = pl.BlockSpec((_BR, WD), lambda i: (i, 0))
_deg1_spec = pl.BlockSpec((_BR, WD), lambda i: (_PB + i, 0))
_agg0_spec = pl.BlockSpec((_BR, DP), lambda i: (i, 0))
_agg1_spec = pl.BlockSpec((_BR, DP), lambda i: (_PB + i, 0))
_row_spec = pl.BlockSpec((_BR, DP), lambda i: (i, 0))


def _dinv_of(d0_ref, d1_ref):
    return lax.rsqrt(d0_ref[:, 0:1] + d1_ref[:, 0:1] + 1.0)


def _agg_of(a0_ref, a1_ref, g_ref):
    return (a0_ref[...].astype(jnp.float32) + a1_ref[...].astype(jnp.float32)
            + g_ref[...].astype(jnp.float32))


def _tcmm_body(x_ref, w_ref, o_ref):
    o_ref[...] = jnp.dot(x_ref[...], w_ref[...],
                         preferred_element_type=jnp.float32)


def _tc1_body(d0_ref, d1_ref, h_ref, o_ref):
    dinv = _dinv_of(d0_ref, d1_ref)
    o_ref[...] = (h_ref[...] * dinv).astype(jnp.bfloat16)


def _tc2_body(d0_ref, d1_ref, a0_ref, a1_ref, g_ref, w_ref, b_ref, o_ref):
    dinv = _dinv_of(d0_ref, d1_ref)
    z = jnp.maximum(_agg_of(a0_ref, a1_ref, g_ref) * dinv + b_ref[...], 0.0)
    o_ref[...] = (jnp.dot(z, w_ref[...], preferred_element_type=jnp.float32)
                  * dinv).astype(jnp.bfloat16)


def _tc3_body(d0_ref, d1_ref, a0_ref, a1_ref, g_ref, b_ref, o_ref):
    dinv = _dinv_of(d0_ref, d1_ref)
    o_ref[...] = (_agg_of(a0_ref, a1_ref, g_ref) * dinv + b_ref[...])[:, :25]


def _tcmm(x, w1p):
    return pl.pallas_call(
        _tcmm_body,
        grid=(_NB,),
        in_specs=[pl.BlockSpec((_BR, 128), lambda i: (i, 0)),
                  pl.BlockSpec((128, DP), lambda i: (0, 0))],
        out_specs=_row_spec,
        out_shape=jax.ShapeDtypeStruct((N, DP), jnp.float32),
    )(x, w1p)


def _tc1(degp, h):
    return pl.pallas_call(
        _tc1_body,
        grid=(_NB,),
        in_specs=[_deg0_spec, _deg1_spec, _row_spec],
        out_specs=_row_spec,
        out_shape=jax.ShapeDtypeStruct((N, DP), jnp.bfloat16),
    )(degp, degp, h)


def _tc2(degp, aggp, g1, w2p, b1p):
    return pl.pallas_call(
        _tc2_body,
        grid=(_NB,),
        in_specs=[_deg0_spec, _deg1_spec, _agg0_spec, _agg1_spec, _row_spec,
                  pl.BlockSpec((DP, DP), lambda i: (0, 0)),
                  pl.BlockSpec((1, DP), lambda i: (0, 0))],
        out_specs=_row_spec,
        out_shape=jax.ShapeDtypeStruct((N, DP), jnp.bfloat16),
    )(degp, degp, aggp, aggp, g1, w2p, b1p)


def _tc3(degp, aggp, g2, b2p):
    return pl.pallas_call(
        _tc3_body,
        grid=(_NB,),
        in_specs=[_deg0_spec, _deg1_spec, _agg0_spec, _agg1_spec, _row_spec,
                  pl.BlockSpec((1, DP), lambda i: (0, 0))],
        out_specs=pl.BlockSpec((_BR, 25), lambda i: (i, 0)),
        out_shape=jax.ShapeDtypeStruct((N, 25), jnp.float32),
    )(degp, degp, aggp, aggp, g2, b2p)


def kernel(x, edge_index, W1, b1, W2, b2):
    ei = edge_index.astype(jnp.int32)
    srcI = ei[0].reshape(NW, NCHUNK, K)
    dstI = ei[1].reshape(NW, NCHUNK, K)

    w1p = jnp.zeros((128, DP), jnp.float32).at[:, :30].set(W1)
    b1p = jnp.zeros((1, DP), jnp.float32).at[0, :30].set(b1)
    w2p = jnp.zeros((DP, DP), jnp.float32).at[:30, :25].set(W2)
    b2p = jnp.zeros((1, DP), jnp.float32).at[0, :25].set(b2)

    zdeg = jnp.zeros((RPG, WD), jnp.float32)
    zagg = jnp.zeros((RPG, DP), jnp.bfloat16)
    ones16 = jnp.ones((K, WD), jnp.float32)
    ridx = (jnp.arange(NS, dtype=jnp.int32)[:, None, None] * RPM
            + jnp.arange(RPM // KM, dtype=jnp.int32)[None, :, None] * KM
            + jnp.arange(KM, dtype=jnp.int32)[None, None, :])

    sc_deg, sc_agg = _sc_kernels()
    h1 = _tcmm(x, w1p)
    degp = sc_deg(dstI, zdeg, ones16, ridx)
    g1 = _tc1(degp, h1)
    a1 = sc_agg(g1, srcI, dstI, zagg, ridx)
    g2 = _tc2(degp, a1, g1, w2p, b1p)
    a2 = sc_agg(g2, srcI, dstI, zagg, ridx)
    return _tc3(degp, a2, g2, b2p)


# speedup vs baseline: 1.4165x; 1.0075x over previous
import functools

import jax
import jax.numpy as jnp
from jax import lax
from jax.experimental import pallas as pl
from jax.experimental.pallas import tpu as pltpu
from jax.experimental.pallas import tpu_sc as plsc

N = 10000
E = 320000
DP = 32
WD = 16
NC = 2
NS = 16
NW = NC * NS
K = 125
NCHUNK = 80
RING = 4
NPAD = 12000
NG = 4
RPG = NPAD // NG
RPM = NPAD // NS
KM = 125


def _merge_planes(acc, ridx_v, mbuf, s):
    base = s * RPM
    for p in range(1, NG):
        pltpu.sync_copy(acc.at[pl.ds(p * NPAD + base, RPM)], mbuf)
        for cch in range(RPM // KM):
            pltpu.sync_copy(mbuf.at[pl.ds(cch * KM, KM)],
                            acc.at[ridx_v.at[cch]], add=True)


def _sc_deg_body(dst_hbm, zeros_hbm, ones_hbm, ridx_hbm, out_hbm,
                 dst_v, ones_v, ridx_v, mbuf, acc, dsem):
    c = lax.axis_index("c")
    s = lax.axis_index("s")
    wid = s * NC + c
    own = (s // NG) * NPAD + (s % NG) * RPG
    pltpu.sync_copy(dst_hbm.at[wid], dst_v)
    pltpu.sync_copy(ones_hbm, ones_v)
    pltpu.sync_copy(ridx_hbm.at[s], ridx_v)
    pltpu.sync_copy(zeros_hbm, acc.at[pl.ds(own, RPG)])
    plsc.subcore_barrier()

    plane = acc.at[pl.ds((s // NG) * NPAD, NPAD)]

    def step(j, carry):
        pltpu.async_copy(ones_v, plane.at[dst_v.at[j]], dsem, add=True)

        @pl.when(j >= 8)
        def _():
            pltpu.make_async_copy(ones_v, plane.at[dst_v.at[j - 8]], dsem).wait()

        return carry

    lax.fori_loop(0, NCHUNK, step, 0)

    def drain(j, carry):
        pltpu.make_async_copy(ones_v, plane.at[dst_v.at[j]], dsem).wait()
        return carry

    lax.fori_loop(NCHUNK - 8, NCHUNK, drain, 0)
    plsc.subcore_barrier()
    _merge_planes(acc, ridx_v, mbuf, s)
    plsc.subcore_barrier()
    pltpu.sync_copy(acc.at[pl.ds(s * RPM, RPM)],
                    out_hbm.at[pl.ds(c * NPAD + s * RPM, RPM)])


def _sc_agg_body(g_hbm, src_hbm, dst_hbm, zeros_hbm, ridx_hbm, out_hbm,
                 src_v, dst_v, ridx_v, rows, mbuf, acc, gsems, ssems):
    c = lax.axis_index("c")
    s = lax.axis_index("s")
    wid = s * NC + c
    own = (s // NG) * NPAD + (s % NG) * RPG
    pltpu.sync_copy(src_hbm.at[wid], src_v)
    pltpu.sync_copy(dst_hbm.at[wid], dst_v)
    pltpu.sync_copy(ridx_hbm.at[s], ridx_v)
    pltpu.sync_copy(zeros_hbm, acc.at[pl.ds(own, RPG)])
    plsc.subcore_barrier()

    plane = acc.at[pl.ds((s // NG) * NPAD, NPAD)]
    for b in range(RING):
        pltpu.async_copy(g_hbm.at[src_v.at[b]], rows.at[b], gsems.at[b])

    def step(i, carry):
        j0 = i * RING
        for b in range(RING):
            j = j0 + b
            pltpu.make_async_copy(g_hbm.at[src_v.at[j]], rows.at[b],
                                  gsems.at[b]).wait()
            pltpu.async_copy(rows.at[b], plane.at[dst_v.at[j]], ssems.at[b],
                             add=True)
        for b in range(RING):
            j = j0 + b

            @pl.when(j + RING < NCHUNK)
            def _():
                pltpu.make_async_copy(rows.at[b], plane.at[dst_v.at[j]],
                                      ssems.at[b]).wait()
                pltpu.async_copy(g_hbm.at[src_v.at[j + RING]], rows.at[b],
                                 gsems.at[b])

        return carry

    lax.fori_loop(0, NCHUNK // RING, step, 0)
    for b in range(RING):
        pltpu.make_async_copy(rows.at[b], plane.at[dst_v.at[NCHUNK - RING + b]],
                              ssems.at[b]).wait()
    plsc.subcore_barrier()
    _merge_planes(acc, ridx_v, mbuf, s)
    plsc.subcore_barrier()
    pltpu.sync_copy(acc.at[pl.ds(s * RPM, RPM)],
                    out_hbm.at[pl.ds(c * NPAD + s * RPM, RPM)])


@functools.cache
def _sc_kernels():
    mesh = plsc.VectorSubcoreMesh(core_axis_name="c", subcore_axis_name="s")
    params = pltpu.CompilerParams(use_tc_tiling_on_sc=False)
    sc_deg = pl.kernel(
        _sc_deg_body,
        out_type=jax.ShapeDtypeStruct((NC * NPAD, WD), jnp.float32),
        mesh=mesh,
        compiler_params=params,
        scratch_types=[
            pltpu.VMEM((NCHUNK, K), jnp.int32),
            pltpu.VMEM((K, WD), jnp.float32),
            pltpu.VMEM((RPM // KM, KM), jnp.int32),
            pltpu.VMEM((RPM, WD), jnp.float32),
            pltpu.VMEM_SHARED((NG * NPAD, WD), jnp.float32),
            pltpu.SemaphoreType.DMA,
        ],
    )
    sc_agg = pl.kernel(
        _sc_agg_body,
        out_type=jax.ShapeDtypeStruct((NC * NPAD, DP), jnp.bfloat16),
        mesh=mesh,
        compiler_params=params,
        scratch_types=[
            pltpu.VMEM((NCHUNK, K), jnp.int32),
            pltpu.VMEM((NCHUNK, K), jnp.int32),
            pltpu.VMEM((RPM // KM, KM), jnp.int32),
            pltpu.VMEM((RING, K, DP), jnp.bfloat16),
            pltpu.VMEM((RPM, DP), jnp.bfloat16),
            pltpu.VMEM_SHARED((NG * NPAD, DP), jnp.bfloat16),
            pltpu.SemaphoreType.DMA((RING,)),
            pltpu.SemaphoreType.DMA((RING,)),
        ],
    )
    return sc_deg, sc_agg


_NB = 2
_BR = N // _NB
_PB = NPAD // _BR

_deg0_spec = pl.BlockSpec((_BR, WD), lambda i: (i, 0))
_deg1_spec = pl.BlockSpec((_BR, WD), lambda i: (_PB + i, 0))
_agg0_spec = pl.BlockSpec((_BR, DP), lambda i: (i, 0))
_agg1_spec = pl.BlockSpec((_BR, DP), lambda i: (_PB + i, 0))
_row_spec = pl.BlockSpec((_BR, DP), lambda i: (i, 0))


def _dinv_of(d0_ref, d1_ref):
    return lax.rsqrt(d0_ref[:, 0:1] + d1_ref[:, 0:1] + 1.0)


def _agg_of(a0_ref, a1_ref, g_ref):
    return (a0_ref[...].astype(jnp.float32) + a1_ref[...].astype(jnp.float32)
            + g_ref[...].astype(jnp.float32))


def _tcmm_body(x_ref, w_ref, o_ref):
    o_ref[...] = jnp.dot(x_ref[...], w_ref[...],
                         preferred_element_type=jnp.float32)


def _tc1_body(d0_ref, d1_ref, h_ref, o_ref):
    dinv = _dinv_of(d0_ref, d1_ref)
    o_ref[...] = (h_ref[...] * dinv).astype(jnp.bfloat16)


def _tc2_body(d0_ref, d1_ref, a0_ref, a1_ref, g_ref, w_ref, b_ref, o_ref):
    dinv = _dinv_of(d0_ref, d1_ref)
    z = jnp.maximum(_agg_of(a0_ref, a1_ref, g_ref) * dinv + b_ref[...], 0.0)
    o_ref[...] = (jnp.dot(z, w_ref[...], preferred_element_type=jnp.float32)
                  * dinv).astype(jnp.bfloat16)


def _tc3_body(d0_ref, d1_ref, a0_ref, a1_ref, g_ref, b_ref, o_ref):
    dinv = _dinv_of(d0_ref, d1_ref)
    o_ref[...] = (_agg_of(a0_ref, a1_ref, g_ref) * dinv + b_ref[...])[:, :25]


def _tcmm(x, w1p):
    return pl.pallas_call(
        _tcmm_body,
        grid=(_NB,),
        in_specs=[pl.BlockSpec((_BR, 128), lambda i: (i, 0)),
                  pl.BlockSpec((128, DP), lambda i: (0, 0))],
        out_specs=_row_spec,
        out_shape=jax.ShapeDtypeStruct((N, DP), jnp.float32),
    )(x, w1p)


def _tc1(degp, h):
    return pl.pallas_call(
        _tc1_body,
        grid=(_NB,),
        in_specs=[_deg0_spec, _deg1_spec, _row_spec],
        out_specs=_row_spec,
        out_shape=jax.ShapeDtypeStruct((N, DP), jnp.bfloat16),
    )(degp, degp, h)


def _tc2(degp, aggp, g1, w2p, b1p):
    return pl.pallas_call(
        _tc2_body,
        grid=(_NB,),
        in_specs=[_deg0_spec, _deg1_spec, _agg0_spec, _agg1_spec, _row_spec,
                  pl.BlockSpec((DP, DP), lambda i: (0, 0)),
                  pl.BlockSpec((1, DP), lambda i: (0, 0))],
        out_specs=_row_spec,
        out_shape=jax.ShapeDtypeStruct((N, DP), jnp.bfloat16),
    )(degp, degp, aggp, aggp, g1, w2p, b1p)


def _tc3(degp, aggp, g2, b2p):
    return pl.pallas_call(
        _tc3_body,
        grid=(_NB,),
        in_specs=[_deg0_spec, _deg1_spec, _agg0_spec, _agg1_spec, _row_spec,
                  pl.BlockSpec((1, DP), lambda i: (0, 0))],
        out_specs=pl.BlockSpec((_BR, 25), lambda i: (i, 0)),
        out_shape=jax.ShapeDtypeStruct((N, 25), jnp.float32),
    )(degp, degp, aggp, aggp, g2, b2p)


def kernel(x, edge_index, W1, b1, W2, b2):
    ei = edge_index.astype(jnp.int32)
    srcI = ei[0].reshape(NW, NCHUNK, K)
    dstI = ei[1].reshape(NW, NCHUNK, K)

    w1p = jnp.zeros((128, DP), jnp.float32).at[:, :30].set(W1)
    b1p = jnp.zeros((1, DP), jnp.float32).at[0, :30].set(b1)
    w2p = jnp.zeros((DP, DP), jnp.float32).at[:30, :25].set(W2)
    b2p = jnp.zeros((1, DP), jnp.float32).at[0, :25].set(b2)

    zdeg = jnp.zeros((RPG, WD), jnp.float32)
    zagg = jnp.zeros((RPG, DP), jnp.bfloat16)
    ones16 = jnp.ones((K, WD), jnp.float32)
    ridx = (jnp.arange(NS, dtype=jnp.int32)[:, None, None] * RPM
            + jnp.arange(RPM // KM, dtype=jnp.int32)[None, :, None] * KM
            + jnp.arange(KM, dtype=jnp.int32)[None, None, :])

    sc_deg, sc_agg = _sc_kernels()
    h1 = _tcmm(x, w1p)
    degp = sc_deg(dstI, zdeg, ones16, ridx)
    g1 = _tc1(degp, h1)
    a1 = sc_agg(g1, srcI, dstI, zagg, ridx)
    g2 = _tc2(degp, a1, g1, w2p, b1p)
    a2 = sc_agg(g2, srcI, dstI, zagg, ridx)
    return _tc3(degp, a2, g2, b2p)
